# initial kernel scaffold (unmeasured)
import jax
import jax.numpy as jnp
from jax import lax
from jax.experimental import pallas as pl
from jax.experimental.pallas import tpu as pltpu

N_DEV = 4
T = 4096
D = 2048
V_SHARD = 8192
C = T // N_DEV


def kernel(ids, E):
    my = lax.axis_index("i")
    loc = ids - my * V_SHARD
    mask = (loc >= 0) & (loc < V_SHARD)
    rows = jnp.take(E, jnp.clip(loc, 0, V_SHARD - 1), axis=0)
    partial = jnp.where(mask[:, None], rows, 0.0).astype(jnp.bfloat16)
    return _ring_allreduce(partial)


def _ring_allreduce(x):

    def body(x_ref, out_ref, rs_recv, rs_stage, rs_send_sems, rs_recv_sems,
             ag_send_sems, ag_recv_sems):
        my = lax.axis_index("i")
        left = lax.rem(my + N_DEV - 1, N_DEV)
        right = lax.rem(my + 1, N_DEV)

        barrier_sem = pltpu.get_barrier_semaphore()
        for nbr in [left, right]:
            pl.semaphore_signal(
                barrier_sem, inc=1,
                device_id=(nbr,), device_id_type=pl.DeviceIdType.MESH,
            )
        pl.semaphore_wait(barrier_sem, 2)

        def chunk(ref, c):
            return ref.at[pl.ds(c * C, C), :]

        rs_stage[0] = x_ref[pl.ds(my * C, C), :]
        for s in range(N_DEV - 1):
            rdma = pltpu.make_async_remote_copy(
                src_ref=rs_stage.at[s % 2],
                dst_ref=rs_recv.at[s],
                send_sem=rs_send_sems.at[s],
                recv_sem=rs_recv_sems.at[s],
                device_id=(right,),
                device_id_type=pl.DeviceIdType.MESH,
            )
            rdma.start()
            rdma.wait()
            rc = lax.rem(my - s - 1 + N_DEV, N_DEV)
            acc = rs_recv[s] + x_ref[pl.ds(rc * C, C), :]
            if s < N_DEV - 2:
                rs_stage[(s + 1) % 2] = acc
            else:
                out_ref[pl.ds(rc * C, C), :] = acc

        for h in range(N_DEV - 1):
            gc = lax.rem(my + 1 - h + N_DEV, N_DEV)
            rdma = pltpu.make_async_remote_copy(
                src_ref=chunk(out_ref, gc),
                dst_ref=chunk(out_ref, gc),
                send_sem=ag_send_sems.at[h],
                recv_sem=ag_recv_sems.at[h],
                device_id=(right,),
                device_id_type=pl.DeviceIdType.MESH,
            )
            rdma.start()
            rdma.wait()

    return pl.pallas_call(
        body,
        out_shape=jax.ShapeDtypeStruct((T, D), jnp.bfloat16),
        in_specs=[pl.BlockSpec(memory_space=pltpu.VMEM)],
        out_specs=pl.BlockSpec(memory_space=pltpu.VMEM),
        scratch_shapes=[
            pltpu.VMEM((N_DEV - 1, C, D), jnp.bfloat16),
            pltpu.VMEM((2, C, D), jnp.bfloat16),
            pltpu.SemaphoreType.DMA((N_DEV - 1,)),
            pltpu.SemaphoreType.DMA((N_DEV - 1,)),
            pltpu.SemaphoreType.DMA((N_DEV - 1,)),
            pltpu.SemaphoreType.DMA((N_DEV - 1,)),
        ],
        compiler_params=pltpu.CompilerParams(collective_id=0),
    )(x)


# baseline (device time: 598722 ns/iter reference)
import jax
import jax.numpy as jnp
from jax import lax
from jax.experimental import pallas as pl
from jax.experimental.pallas import tpu as pltpu

N_DEV = 4
T = 4096
D = 2048
V_SHARD = 8192
C = T // N_DEV


def kernel(ids, E):
    my = lax.axis_index("i")
    loc = ids - my * V_SHARD
    mask = (loc >= 0) & (loc < V_SHARD)
    rows = jnp.take(E, jnp.clip(loc, 0, V_SHARD - 1), axis=0)
    partial = jnp.where(mask[:, None], rows, 0.0).astype(jnp.bfloat16)
    return _ring_allreduce(partial)


def _ring_allreduce(x):

    def body(x_ref, out_ref, rs_recv, rs_stage, rs_send_sems, rs_recv_sems,
             ag_send_sems, ag_recv_sems):
        my = lax.axis_index("i")
        left = lax.rem(my + N_DEV - 1, N_DEV)
        right = lax.rem(my + 1, N_DEV)

        barrier_sem = pltpu.get_barrier_semaphore()
        for nbr in [left, right]:
            pl.semaphore_signal(
                barrier_sem, inc=1,
                device_id=(nbr,), device_id_type=pl.DeviceIdType.MESH,
            )
        pl.semaphore_wait(barrier_sem, 2)

        def chunk(ref, c):
            return ref.at[pl.ds(c * C, C), :]

        rs_stage[0] = x_ref[pl.ds(my * C, C), :]
        for s in range(N_DEV - 1):
            rdma = pltpu.make_async_remote_copy(
                src_ref=rs_stage.at[s % 2],
                dst_ref=rs_recv.at[s],
                send_sem=rs_send_sems.at[s],
                recv_sem=rs_recv_sems.at[s],
                device_id=(right,),
                device_id_type=pl.DeviceIdType.MESH,
            )
            rdma.start()
            rdma.wait()
            rc = lax.rem(my - s - 1 + N_DEV, N_DEV)
            acc = rs_recv[s] + x_ref[pl.ds(rc * C, C), :]
            if s < N_DEV - 2:
                rs_stage[(s + 1) % 2] = acc
            else:
                out_ref[pl.ds(rc * C, C), :] = acc

        for h in range(N_DEV - 1):
            gc = lax.rem(my + 1 - h + N_DEV, N_DEV)
            rdma = pltpu.make_async_remote_copy(
                src_ref=chunk(out_ref, gc),
                dst_ref=chunk(out_ref, gc),
                send_sem=ag_send_sems.at[h],
                recv_sem=ag_recv_sems.at[h],
                device_id=(right,),
                device_id_type=pl.DeviceIdType.MESH,
            )
            rdma.start()
            rdma.wait()

    return pl.pallas_call(
        body,
        out_shape=jax.ShapeDtypeStruct((T, D), jnp.bfloat16),
        in_specs=[pl.BlockSpec(memory_space=pltpu.VMEM)],
        out_specs=pl.BlockSpec(memory_space=pltpu.VMEM),
        scratch_shapes=[
            pltpu.VMEM((N_DEV - 1, C, D), jnp.bfloat16),
            pltpu.VMEM((2, C, D), jnp.bfloat16),
            pltpu.SemaphoreType.DMA((N_DEV - 1,)),
            pltpu.SemaphoreType.DMA((N_DEV - 1,)),
            pltpu.SemaphoreType.DMA((N_DEV - 1,)),
            pltpu.SemaphoreType.DMA((N_DEV - 1,)),
        ],
        compiler_params=pltpu.CompilerParams(
            collective_id=0,
            vmem_limit_bytes=100 * 1024 * 1024,
        ),
    )(x)


# device time: 303190 ns/iter; 1.9747x vs baseline; 1.9747x over previous
import jax
import jax.numpy as jnp
from jax import lax
from jax.experimental import pallas as pl
from jax.experimental.pallas import tpu as pltpu

N_DEV = 4
T = 4096
D = 2048
HD = D // 2
V_SHARD = 8192
C = T // N_DEV
GATHER_WINDOW = 512


def kernel(ids, E):
    def body(ids_smem, e_hbm, out_hbm, g_ref, rs_recv, accbuf, zrow,
             gsem, outsem, rs_send_sems, rs_recv_sems,
             ag_send_sems, ag_recv_sems):
        my = lax.axis_index("i")
        left = lax.rem(my + N_DEV - 1, N_DEV)
        right = lax.rem(my + 1, N_DEV)
        base = my * V_SHARD

        zrow[...] = jnp.zeros_like(zrow)

        def gstep(t, carry):
            @pl.when(t < T)
            def _():
                loc = ids_smem[t] - base
                owned = (loc >= 0) & (loc < V_SHARD)
                locc = jnp.clip(loc, 0, V_SHARD - 1)

                @pl.when(owned)
                def _():
                    pltpu.make_async_copy(
                        e_hbm.at[pl.ds(locc, 1), :],
                        g_ref.at[pl.ds(t, 1), :],
                        gsem,
                    ).start()

                @pl.when(jnp.logical_not(owned))
                def _():
                    pltpu.make_async_copy(
                        zrow.at[pl.ds(0, 1), :],
                        g_ref.at[pl.ds(t, 1), :],
                        gsem,
                    ).start()

            @pl.when(t >= GATHER_WINDOW)
            def _():
                pltpu.make_async_copy(
                    zrow.at[pl.ds(0, 1), :],
                    g_ref.at[pl.ds(t - GATHER_WINDOW, 1), :],
                    gsem,
                ).wait()

            return carry

        lax.fori_loop(0, T + GATHER_WINDOW, gstep, 0)

        barrier_sem = pltpu.get_barrier_semaphore()
        for nbr in [left, right]:
            pl.semaphore_signal(
                barrier_sem, inc=1,
                device_id=(nbr,), device_id_type=pl.DeviceIdType.MESH,
            )
        pl.semaphore_wait(barrier_sem, 2)

        def xval(c, d):
            return g_ref[pl.ds(c * C, C), pl.ds(d * HD, HD)].astype(jnp.bfloat16)

        dests = (right, left)

        def rchunk(d, s):
            step = (s + 1) if d else -(s + 1)
            return lax.rem(my + step + 2 * N_DEV, N_DEV)

        accbuf[0] = xval(my, 0)
        accbuf[1] = xval(my, 1)
        for s in range(N_DEV - 1):
            rdmas = []
            for d in (0, 1):
                rdma = pltpu.make_async_remote_copy(
                    src_ref=accbuf.at[d] if s == 0 else rs_recv.at[d, s - 1],
                    dst_ref=rs_recv.at[d, s],
                    send_sem=rs_send_sems.at[d, s],
                    recv_sem=rs_recv_sems.at[d, s],
                    device_id=(dests[d],),
                    device_id_type=pl.DeviceIdType.MESH,
                )
                rdma.start()
                rdmas.append(rdma)
            for d, rdma in enumerate(rdmas):
                rdma.wait()
                rc = rchunk(d, s)
                acc = rs_recv[d, s] + xval(rc, d)
                if s < N_DEV - 2:
                    rs_recv[d, s] = acc
                else:
                    accbuf[d] = acc
                    pltpu.make_async_copy(
                        accbuf.at[d],
                        out_hbm.at[pl.ds(rc * C, C), pl.ds(d * HD, HD)],
                        outsem,
                    ).start()
        for d in (0, 1):
            pltpu.make_async_copy(
                accbuf.at[d],
                out_hbm.at[pl.ds(rchunk(d, N_DEV - 2) * C, C),
                           pl.ds(d * HD, HD)],
                outsem,
            ).wait()

        for h in range(N_DEV - 1):
            rdmas = []
            for d in (0, 1):
                gc = lax.rem(my + (h - 1 if d else 1 - h) + 2 * N_DEV, N_DEV)
                sl = out_hbm.at[pl.ds(gc * C, C), pl.ds(d * HD, HD)]
                rdma = pltpu.make_async_remote_copy(
                    src_ref=sl,
                    dst_ref=sl,
                    send_sem=ag_send_sems.at[d, h],
                    recv_sem=ag_recv_sems.at[d, h],
                    device_id=(dests[d],),
                    device_id_type=pl.DeviceIdType.MESH,
                )
                rdma.start()
                rdmas.append(rdma)
            for rdma in rdmas:
                rdma.wait()

    return pl.pallas_call(
        body,
        out_shape=jax.ShapeDtypeStruct((T, D), jnp.bfloat16),
        in_specs=[
            pl.BlockSpec(memory_space=pltpu.SMEM),
            pl.BlockSpec(memory_space=pl.ANY),
        ],
        out_specs=pl.BlockSpec(memory_space=pl.ANY),
        scratch_shapes=[
            pltpu.VMEM((T, D), jnp.float32),
            pltpu.VMEM((2, N_DEV - 1, C, HD), jnp.bfloat16),
            pltpu.VMEM((2, C, HD), jnp.bfloat16),
            pltpu.VMEM((8, D), jnp.float32),
            pltpu.SemaphoreType.DMA,
            pltpu.SemaphoreType.DMA,
            pltpu.SemaphoreType.DMA((2, N_DEV - 1)),
            pltpu.SemaphoreType.DMA((2, N_DEV - 1)),
            pltpu.SemaphoreType.DMA((2, N_DEV - 1)),
            pltpu.SemaphoreType.DMA((2, N_DEV - 1)),
        ],
        compiler_params=pltpu.CompilerParams(
            collective_id=0,
            vmem_limit_bytes=60 * 1024 * 1024,
        ),
    )(ids, E)


# device time: 192299 ns/iter; 3.1135x vs baseline; 1.5767x over previous
import jax
import jax.numpy as jnp
from jax import lax
from jax.experimental import pallas as pl
from jax.experimental.pallas import tpu as pltpu

N_DEV = 4
T = 4096
D = 2048
HD = D // 2
V_SHARD = 8192
C = T // N_DEV

_ORDER_OF_REL = (0, 2, 3, 1)


def kernel(ids, E):
    my = lax.axis_index("i")
    tchunk = jax.lax.iota(jnp.int32, T) // C
    rel = jnp.remainder(tchunk - my, N_DEV)
    order = jnp.take(jnp.asarray(_ORDER_OF_REL, jnp.int32), rel)
    owner = ids // V_SHARD
    key = jnp.where(owner == my, order, N_DEV).astype(jnp.int32)
    perm = jnp.argsort(key, stable=True).astype(jnp.int32)
    loc_sorted = jnp.take(ids - my * V_SHARD, perm).astype(jnp.int32)
    counts = jnp.sum(
        key[None, :] == jnp.arange(N_DEV, dtype=jnp.int32)[:, None], axis=1
    ).astype(jnp.int32)
    prefix = jnp.concatenate(
        [jnp.zeros((1,), jnp.int32), jnp.cumsum(counts)[:-1]]
    ).astype(jnp.int32)

    def body(perm_smem, loc_smem, counts_smem, prefix_smem, ids_v, e_hbm,
             out_hbm, g_ref, rs_recv, accbuf,
             gsem, outsem, rs_send_sems, rs_recv_sems,
             ag_send_sems, ag_recv_sems):
        my = lax.axis_index("i")
        left = lax.rem(my + N_DEV - 1, N_DEV)
        right = lax.rem(my + 1, N_DEV)
        base = my * V_SHARD

        def issue_group(o):
            s0 = prefix_smem[o]
            n = counts_smem[o]

            def st(i, carry):
                t = perm_smem[s0 + i]
                l = loc_smem[s0 + i]
                pltpu.make_async_copy(
                    e_hbm.at[pl.ds(l, 1), :],
                    g_ref.at[pl.ds(t, 1), :],
                    gsem.at[o],
                ).start()
                return carry

            lax.fori_loop(0, n, st, 0)

        def drain_group(o):
            n = counts_smem[o]

            def wt(i, carry):
                pltpu.make_async_copy(
                    e_hbm.at[pl.ds(0, 1), :],
                    g_ref.at[pl.ds(0, 1), :],
                    gsem.at[o],
                ).wait()
                return carry

            lax.fori_loop(0, n, wt, 0)

        def xval(c, d):
            idv = ids_v[pl.ds(c * C, C), :]
            ok = (idv >= base) & (idv < base + V_SHARD)
            gg = g_ref[pl.ds(c * C, C), pl.ds(d * HD, HD)]
            return jnp.where(ok, gg, 0.0).astype(jnp.bfloat16)

        dests = (right, left)

        def rchunk(d, s):
            step = (s + 1) if d else -(s + 1)
            return lax.rem(my + step + 2 * N_DEV, N_DEV)

        issue_group(0)
        drain_group(0)
        accbuf[0] = xval(my, 0)
        accbuf[1] = xval(my, 1)

        barrier_sem = pltpu.get_barrier_semaphore()
        for nbr in [left, right]:
            pl.semaphore_signal(
                barrier_sem, inc=1,
                device_id=(nbr,), device_id_type=pl.DeviceIdType.MESH,
            )
        pl.semaphore_wait(barrier_sem, 2)

        for s in range(N_DEV - 1):
            rdmas = []
            for d in (0, 1):
                rdma = pltpu.make_async_remote_copy(
                    src_ref=accbuf.at[d] if s == 0 else rs_recv.at[d, s - 1],
                    dst_ref=rs_recv.at[d, s],
                    send_sem=rs_send_sems.at[d, s],
                    recv_sem=rs_recv_sems.at[d, s],
                    device_id=(dests[d],),
                    device_id_type=pl.DeviceIdType.MESH,
                )
                rdma.start()
                rdmas.append(rdma)
            if s == 0:
                issue_group(1)
                issue_group(2)
                drain_group(1)
                drain_group(2)
            elif s == 1:
                issue_group(3)
                drain_group(3)
            for d, rdma in enumerate(rdmas):
                rdma.wait()
                rc = rchunk(d, s)
                acc = rs_recv[d, s] + xval(rc, d)
                if s < N_DEV - 2:
                    rs_recv[d, s] = acc
                else:
                    accbuf[d] = acc
                    pltpu.make_async_copy(
                        accbuf.at[d],
                        out_hbm.at[pl.ds(rc * C, C), pl.ds(d * HD, HD)],
                        outsem,
                    ).start()
        for d in (0, 1):
            pltpu.make_async_copy(
                accbuf.at[d],
                out_hbm.at[pl.ds(rchunk(d, N_DEV - 2) * C, C),
                           pl.ds(d * HD, HD)],
                outsem,
            ).wait()

        for h in range(N_DEV - 1):
            rdmas = []
            for d in (0, 1):
                gc = lax.rem(my + (h - 1 if d else 1 - h) + 2 * N_DEV, N_DEV)
                sl = out_hbm.at[pl.ds(gc * C, C), pl.ds(d * HD, HD)]
                rdma = pltpu.make_async_remote_copy(
                    src_ref=sl,
                    dst_ref=sl,
                    send_sem=ag_send_sems.at[d, h],
                    recv_sem=ag_recv_sems.at[d, h],
                    device_id=(dests[d],),
                    device_id_type=pl.DeviceIdType.MESH,
                )
                rdma.start()
                rdmas.append(rdma)
            for rdma in rdmas:
                rdma.wait()

    return pl.pallas_call(
        body,
        out_shape=jax.ShapeDtypeStruct((T, D), jnp.bfloat16),
        in_specs=[
            pl.BlockSpec(memory_space=pltpu.SMEM),
            pl.BlockSpec(memory_space=pltpu.SMEM),
            pl.BlockSpec(memory_space=pltpu.SMEM),
            pl.BlockSpec(memory_space=pltpu.SMEM),
            pl.BlockSpec(memory_space=pltpu.VMEM),
            pl.BlockSpec(memory_space=pl.ANY),
        ],
        out_specs=pl.BlockSpec(memory_space=pl.ANY),
        scratch_shapes=[
            pltpu.VMEM((T, D), jnp.float32),
            pltpu.VMEM((2, N_DEV - 1, C, HD), jnp.bfloat16),
            pltpu.VMEM((2, C, HD), jnp.bfloat16),
            pltpu.SemaphoreType.DMA((N_DEV,)),
            pltpu.SemaphoreType.DMA,
            pltpu.SemaphoreType.DMA((2, N_DEV - 1)),
            pltpu.SemaphoreType.DMA((2, N_DEV - 1)),
            pltpu.SemaphoreType.DMA((2, N_DEV - 1)),
            pltpu.SemaphoreType.DMA((2, N_DEV - 1)),
        ],
        compiler_params=pltpu.CompilerParams(
            collective_id=0,
            vmem_limit_bytes=60 * 1024 * 1024,
        ),
    )(perm, loc_sorted, counts, prefix, ids.reshape(T, 1), E)
